# Initial kernel scaffold; baseline (speedup 1.0000x reference)
import jax
import jax.numpy as jnp
from jax.experimental import pallas as pl


def kernel(emissions, tags, mask, transitions):
    raise NotImplementedError("write your pallas kernel here")


# speedup vs baseline: 22.7024x; 22.7024x over previous
import functools

import jax
import jax.numpy as jnp
from jax import lax
from jax.experimental import pallas as pl
from jax.experimental.pallas import tpu as pltpu
from jax.experimental.pallas import tpu_sc as plsc

_B, _T, _N = 128, 512, 128
_L = 16
_NW = 32
_TCHUNK = _T // _NW
_TBT = 128


def _sc_body(tagsp_hbm, maskt_hbm, trans_hbm, out_hbm,
             table_v, tags_v, mask_v, res_v):
    wid = lax.axis_index("s") * 2 + lax.axis_index("c")
    t0 = wid * _TCHUNK
    pltpu.sync_copy(trans_hbm, table_v)
    pltpu.sync_copy(tagsp_hbm.at[pl.ds(t0 * _B, (_TCHUNK + 1) * _B)], tags_v)
    pltpu.sync_copy(maskt_hbm.at[pl.ds(t0 * _B, _TCHUNK * _B)], mask_v)

    total = jnp.zeros((_L,), jnp.float32)
    for i in range(_TCHUNK):
        msum = jnp.zeros((_L,), jnp.float32)
        acc = jnp.zeros((_L,), jnp.float32)
        for g in range(_B // _L):
            off = g * _L
            msum = msum + mask_v[pl.ds(i * _B + off, _L)]
            tp = tags_v[pl.ds(i * _B + off, _L)]
            tc = tags_v[pl.ds((i + 1) * _B + off, _L)]
            acc = acc + plsc.load_gather(table_v, [tp * _N + tc])
        wt = jnp.sum(msum)
        wt = jnp.where(t0 + i == 0, 0.0, wt)
        total = total + acc * wt
    res_v[...] = total
    pltpu.sync_copy(res_v, out_hbm.at[pl.ds(wid * _L, _L)])


@jax.jit
def _sc_call(tagsp, maskt, trans_flat):
    mesh = plsc.VectorSubcoreMesh(core_axis_name="c", subcore_axis_name="s")
    f = functools.partial(
        pl.kernel,
        mesh=mesh,
        compiler_params=pltpu.CompilerParams(needs_layout_passes=False),
        out_type=jax.ShapeDtypeStruct((_NW * _L,), jnp.float32),
        scratch_types=[
            pltpu.VMEM((_N * _N,), jnp.float32),
            pltpu.VMEM(((_TCHUNK + 1) * _B,), jnp.int32),
            pltpu.VMEM((_TCHUNK * _B,), jnp.float32),
            pltpu.VMEM((_L,), jnp.float32),
        ],
    )(_sc_body)
    return f(tagsp, maskt, trans_flat)


def _tc_body(emis_ref, mask_ref, tags_ref, trans_ref, out_ref):
    i = pl.program_id(0)
    es = jnp.sum(emis_ref[...], axis=2)
    tglob = i * _TBT + lax.broadcasted_iota(jnp.int32, (_B, _TBT), 1)
    w = jnp.where(tglob == 0, 1.0, mask_ref[...])
    part = jnp.sum(es * w)

    @pl.when(i == 0)
    def _init():
        oh = (tags_ref[:, 0:1]
              == lax.broadcasted_iota(jnp.int32, (_B, _N), 1)).astype(jnp.float32)
        s1 = jnp.sum(jnp.dot(oh, trans_ref[...],
                             preferred_element_type=jnp.float32))
        out_ref[...] = jnp.zeros_like(out_ref) + s1

    out_ref[...] = out_ref[...] + part


@jax.jit
def _tc_call(emissions, mask, tags, transitions):
    return pl.pallas_call(
        _tc_body,
        grid=(_T // _TBT,),
        in_specs=[
            pl.BlockSpec((_B, _TBT, _N), lambda i: (0, i, 0)),
            pl.BlockSpec((_B, _TBT), lambda i: (0, i)),
            pl.BlockSpec((_B, _T), lambda i: (0, 0)),
            pl.BlockSpec((_N, _N), lambda i: (0, 0)),
        ],
        out_specs=pl.BlockSpec((1, 1), lambda i: (0, 0)),
        out_shape=jax.ShapeDtypeStruct((1, 1), jnp.float32),
    )(emissions, mask, tags, transitions)


def kernel(emissions, tags, mask, transitions):
    tagst = tags.T
    tagsp = jnp.concatenate([tagst[:1], tagst], axis=0)
    maskt = mask.T
    sc_part = _sc_call(tagsp.reshape(-1), maskt.reshape(-1),
                       transitions.reshape(-1))
    tc_part = _tc_call(emissions, mask, tags, transitions)
    return tc_part[0, 0] + jnp.sum(sc_part)


# speedup vs baseline: 27.0129x; 1.1899x over previous
import functools

import jax
import jax.numpy as jnp
from jax import lax
from jax.experimental import pallas as pl
from jax.experimental.pallas import tpu as pltpu
from jax.experimental.pallas import tpu_sc as plsc

_B, _T, _N = 128, 512, 128
_L = 16
_NW = 32
_TCHUNK = _T // _NW
_TBT = 128


def _sc_body(tagsp_hbm, maskt_hbm, trans_hbm, out_hbm,
             table_v, tags_v, mask_v, res_v):
    wid = lax.axis_index("s") * 2 + lax.axis_index("c")
    t0 = wid * _TCHUNK
    pltpu.sync_copy(trans_hbm, table_v)
    pltpu.sync_copy(tagsp_hbm.at[pl.ds(t0 * _B, (_TCHUNK + 1) * _B)], tags_v)
    pltpu.sync_copy(maskt_hbm.at[pl.ds(t0 * _B, _TCHUNK * _B)], mask_v)

    total = jnp.zeros((_L,), jnp.float32)
    for i in range(_TCHUNK):
        msum = jnp.zeros((_L,), jnp.float32)
        acc = jnp.zeros((_L,), jnp.float32)
        for g in range(_B // _L):
            off = g * _L
            msum = msum + mask_v[pl.ds(i * _B + off, _L)]
            tp = tags_v[pl.ds(i * _B + off, _L)]
            tc = tags_v[pl.ds((i + 1) * _B + off, _L)]
            acc = acc + plsc.load_gather(table_v, [tp * _N + tc])
        wt = jnp.sum(msum)
        wt = jnp.where(t0 + i == 0, 0.0, wt)
        total = total + acc * wt
    res_v[...] = total
    pltpu.sync_copy(res_v, out_hbm.at[pl.ds(wid * _L, _L)])


@jax.jit
def _sc_call(tagsp, maskt, trans_flat):
    mesh = plsc.VectorSubcoreMesh(core_axis_name="c", subcore_axis_name="s")
    f = functools.partial(
        pl.kernel,
        mesh=mesh,
        compiler_params=pltpu.CompilerParams(needs_layout_passes=False),
        out_type=jax.ShapeDtypeStruct((_NW * _L,), jnp.float32),
        scratch_types=[
            pltpu.VMEM((_N * _N,), jnp.float32),
            pltpu.VMEM(((_TCHUNK + 1) * _B,), jnp.int32),
            pltpu.VMEM((_TCHUNK * _B,), jnp.float32),
            pltpu.VMEM((_L,), jnp.float32),
        ],
    )(_sc_body)
    return f(tagsp, maskt, trans_flat)


_RBLK = 4096


def _tc_body(emis_ref, mask_ref, tags_ref, trans_ref, out_ref):
    i = pl.program_id(0)
    f = i * _RBLK + lax.broadcasted_iota(jnp.int32, (1, _RBLK), 1)
    w = jnp.where(f % _T == 0, 1.0, mask_ref[...])
    c = jnp.dot(w, emis_ref[...],
                preferred_element_type=jnp.float32)

    @pl.when(i == 0)
    def _init():
        oh = (tags_ref[:, 0:1]
              == lax.broadcasted_iota(jnp.int32, (_B, _N), 1)).astype(jnp.float32)
        s1 = jnp.sum(jnp.dot(oh, trans_ref[...],
                             preferred_element_type=jnp.float32))
        out_ref[...] = jnp.where(
            lax.broadcasted_iota(jnp.int32, (1, _N), 1) == 0, s1, 0.0)

    out_ref[...] = out_ref[...] + c


@jax.jit
def _tc_call(emissions, mask, tags, transitions):
    e2 = emissions.reshape(_B * _T, _N)
    m2 = mask.reshape(1, _B * _T)
    return pl.pallas_call(
        _tc_body,
        grid=(_B * _T // _RBLK,),
        in_specs=[
            pl.BlockSpec((_RBLK, _N), lambda i: (i, 0)),
            pl.BlockSpec((1, _RBLK), lambda i: (0, i)),
            pl.BlockSpec((_B, _T), lambda i: (0, 0)),
            pl.BlockSpec((_N, _N), lambda i: (0, 0)),
        ],
        out_specs=pl.BlockSpec((1, _N), lambda i: (0, 0)),
        out_shape=jax.ShapeDtypeStruct((1, _N), jnp.float32),
    )(e2, m2, tags, transitions)


def kernel(emissions, tags, mask, transitions):
    tagst = tags.T
    tagsp = jnp.concatenate([tagst[:1], tagst], axis=0)
    maskt = mask.T
    sc_part = _sc_call(tagsp.reshape(-1), maskt.reshape(-1),
                       transitions.reshape(-1))
    tc_part = _tc_call(emissions, mask, tags, transitions)
    return jnp.sum(tc_part) + jnp.sum(sc_part)
